# baseline (speedup 1.0000x reference)
import functools

import jax
import jax.numpy as jnp
from jax import lax
from jax.experimental import pallas as pl
from jax.experimental.pallas import tpu as pltpu
from jax.experimental.pallas import tpu_sc as plsc

N = 160000
D_IN = 256
D_HID = 512
N_BINS = 10000


_ROWS_BLK = 640


def _mlp_body(x_ref, w1_ref, b1_ref, w2_ref, b2_ref, o_ref):
    h = jnp.dot(x_ref[...], w1_ref[...], preferred_element_type=jnp.float32)
    h = jnp.maximum(h + b1_ref[...], 0.0).astype(jnp.bfloat16)
    y = jnp.dot(h, w2_ref[...], preferred_element_type=jnp.float32)
    o_ref[...] = y + b2_ref[...]


def _mlp(x, W1, b1, W2, b2):
    grid = (N // _ROWS_BLK,)
    return pl.pallas_call(
        _mlp_body,
        grid=grid,
        in_specs=[
            pl.BlockSpec((_ROWS_BLK, D_IN), lambda i: (i, 0)),
            pl.BlockSpec((D_IN, D_HID), lambda i: (0, 0)),
            pl.BlockSpec((1, D_HID), lambda i: (0, 0)),
            pl.BlockSpec((D_HID, D_IN), lambda i: (0, 0)),
            pl.BlockSpec((1, D_IN), lambda i: (0, 0)),
        ],
        out_specs=pl.BlockSpec((_ROWS_BLK, D_IN), lambda i: (i, 0)),
        out_shape=jax.ShapeDtypeStruct((N, D_IN), jnp.float32),
    )(x.astype(jnp.bfloat16), W1.astype(jnp.bfloat16),
      b1.reshape(1, D_HID), W2.astype(jnp.bfloat16), b2.reshape(1, D_IN))


_NC, _NS = 2, 16
_HALF = D_IN // _NC
_ROWS_PER_SUB = N // _NS
_CH = 80
_NCHUNK = _ROWS_PER_SUB // _CH
_EXP_CH = 16
_N_EXP_CHUNKS = N_BINS // _EXP_CH


def _segsum(items, idx32):
    mesh = plsc.VectorSubcoreMesh(
        core_axis_name="c", subcore_axis_name="s",
        num_cores=_NC, num_subcores=_NS,
    )

    @functools.partial(
        pl.kernel,
        out_type=jax.ShapeDtypeStruct((N_BINS, D_IN), jnp.float32),
        mesh=mesh,
        scratch_types=[
            pltpu.VMEM((_CH,), jnp.int32),
            pltpu.VMEM((_CH, _HALF), jnp.float32),
            pltpu.VMEM((_EXP_CH, _HALF), jnp.float32),
            pltpu.VMEM_SHARED((N_BINS, _HALF), jnp.float32),
        ],
    )
    def k(items_hbm, idx_hbm, out_hbm, idx_v, rows_v, stage_v, acc_sh):
        c = lax.axis_index("c")
        s = lax.axis_index("s")
        col0 = c * _HALF

        n_t = (_N_EXP_CHUNKS - s + _NS - 1) // _NS

        zero = jnp.zeros((16,), jnp.float32)
        for i in range(_EXP_CH):
            for j in range(_HALF // 16):
                stage_v[i, pl.ds(j * 16, 16)] = zero

        def zacc(t, carry):
            r0 = (s + t * _NS) * _EXP_CH
            pltpu.sync_copy(stage_v, acc_sh.at[pl.ds(r0, _EXP_CH)])
            return carry

        lax.fori_loop(0, n_t, zacc, 0)
        plsc.subcore_barrier()

        def body(i, carry):
            row0 = s * _ROWS_PER_SUB + i * _CH
            pltpu.sync_copy(idx_hbm.at[pl.ds(row0, _CH)], idx_v)
            pltpu.sync_copy(
                items_hbm.at[pl.ds(row0, _CH), pl.ds(col0, _HALF)], rows_v)
            pltpu.sync_copy(rows_v, acc_sh.at[idx_v], add=True)
            return carry

        lax.fori_loop(0, _NCHUNK, body, 0)
        plsc.subcore_barrier()

        def export(t, carry):
            r0 = (s + t * _NS) * _EXP_CH
            pltpu.sync_copy(acc_sh.at[pl.ds(r0, _EXP_CH)], stage_v)
            pltpu.sync_copy(
                stage_v, out_hbm.at[pl.ds(r0, _EXP_CH), pl.ds(col0, _HALF)])
            return carry

        lax.fori_loop(0, n_t, export, 0)

    return k(items, idx32)


def kernel(x, idxs, n_bins, W1, b1, W2, b2):
    idx32 = jnp.minimum(idxs, N_BINS - 1).astype(jnp.int32)
    items = _mlp(x, W1, b1, W2, b2)
    return _segsum(items, idx32)


# speedup vs baseline: 1.4842x; 1.4842x over previous
import functools

import jax
import jax.numpy as jnp
from jax import lax
from jax.experimental import pallas as pl
from jax.experimental.pallas import tpu as pltpu
from jax.experimental.pallas import tpu_sc as plsc

N = 160000
D_IN = 256
D_HID = 512
N_BINS = 10000


_ROWS_BLK = 640


def _mlp_body(x_ref, w1_ref, b1_ref, w2_ref, b2_ref, o_ref):
    h = jnp.dot(x_ref[...], w1_ref[...], preferred_element_type=jnp.float32)
    h = jnp.maximum(h + b1_ref[...], 0.0)
    y = jnp.dot(h, w2_ref[...], preferred_element_type=jnp.float32)
    o_ref[...] = y + b2_ref[...]


def _mlp(x, W1, b1, W2, b2):
    grid = (N // _ROWS_BLK,)
    return pl.pallas_call(
        _mlp_body,
        grid=grid,
        in_specs=[
            pl.BlockSpec((_ROWS_BLK, D_IN), lambda i: (i, 0)),
            pl.BlockSpec((D_IN, D_HID), lambda i: (0, 0)),
            pl.BlockSpec((1, D_HID), lambda i: (0, 0)),
            pl.BlockSpec((D_HID, D_IN), lambda i: (0, 0)),
            pl.BlockSpec((1, D_IN), lambda i: (0, 0)),
        ],
        out_specs=pl.BlockSpec((_ROWS_BLK, D_IN), lambda i: (i, 0)),
        out_shape=jax.ShapeDtypeStruct((N, D_IN), jnp.float32),
    )(x, W1, b1.reshape(1, D_HID), W2, b2.reshape(1, D_IN))


_NC, _NS = 2, 16
_HALF = D_IN // _NC
_ROWS_PER_SUB = N // _NS
_CH = 80
_NCHUNK = _ROWS_PER_SUB // _CH
_EXP_CH = 80
_N_EXP_CHUNKS = N_BINS // _EXP_CH


def _segsum(items, idx3d):
    mesh = plsc.VectorSubcoreMesh(
        core_axis_name="c", subcore_axis_name="s",
        num_cores=_NC, num_subcores=_NS,
    )

    @functools.partial(
        pl.kernel,
        out_type=jax.ShapeDtypeStruct((N_BINS, D_IN), jnp.float32),
        mesh=mesh,
        scratch_types=[
            pltpu.VMEM((_NCHUNK, _CH), jnp.int32),
            pltpu.VMEM((_CH, _HALF), jnp.float32),
            pltpu.VMEM((_CH, _HALF), jnp.float32),
            pltpu.VMEM((_EXP_CH, _HALF), jnp.float32),
            pltpu.VMEM_SHARED((N_BINS, _HALF), jnp.float32),
            pltpu.SemaphoreType.DMA,
            pltpu.SemaphoreType.DMA,
        ],
    )
    def k(items_hbm, idx_hbm, out_hbm,
          idx_v, rows0, rows1, stage_v, acc_sh, sem0, sem1):
        c = lax.axis_index("c")
        s = lax.axis_index("s")
        col0 = c * _HALF
        row_base = s * _ROWS_PER_SUB

        pltpu.sync_copy(idx_hbm.at[s], idx_v)

        zero = jnp.zeros((16,), jnp.float32)

        def zst(i, carry):
            for j in range(_HALF // 16):
                stage_v[i, pl.ds(j * 16, 16)] = zero
            return carry

        lax.fori_loop(0, _EXP_CH, zst, 0)

        n_t = (_N_EXP_CHUNKS - s + _NS - 1) // _NS

        def zacc(t, carry):
            r0 = (s + t * _NS) * _EXP_CH
            pltpu.sync_copy(stage_v, acc_sh.at[pl.ds(r0, _EXP_CH)])
            return carry

        lax.fori_loop(0, n_t, zacc, 0)
        plsc.subcore_barrier()

        def start(chunk, buf, sem):
            row0 = row_base + chunk * _CH
            pltpu.async_copy(
                items_hbm.at[pl.ds(row0, _CH), pl.ds(col0, _HALF)], buf, sem)

        def wait(buf, sem):
            pltpu.make_async_copy(
                items_hbm.at[pl.ds(row_base, _CH), pl.ds(col0, _HALF)],
                buf, sem).wait()

        def scat(chunk, buf):
            pltpu.sync_copy(buf, acc_sh.at[idx_v.at[chunk]], add=True)

        start(0, rows0, sem0)

        def pair(i, carry):
            c0 = 2 * i
            c1 = c0 + 1
            start(c1, rows1, sem1)
            wait(rows0, sem0)
            scat(c0, rows0)

            @pl.when(c1 + 1 < _NCHUNK)
            def _():
                start(c1 + 1, rows0, sem0)

            wait(rows1, sem1)
            scat(c1, rows1)
            return carry

        lax.fori_loop(0, _NCHUNK // 2, pair, 0)
        wait(rows0, sem0)
        scat(_NCHUNK - 1, rows0)
        plsc.subcore_barrier()

        def export(t, carry):
            r0 = (s + t * _NS) * _EXP_CH
            pltpu.sync_copy(acc_sh.at[pl.ds(r0, _EXP_CH)], stage_v)
            pltpu.sync_copy(
                stage_v, out_hbm.at[pl.ds(r0, _EXP_CH), pl.ds(col0, _HALF)])
            return carry

        lax.fori_loop(0, n_t, export, 0)

    return k(items, idx3d)


def kernel(x, idxs, n_bins, W1, b1, W2, b2):
    idx32 = jnp.minimum(idxs, N_BINS - 1).astype(jnp.int32)
    idx3d = idx32.reshape(_NS, _NCHUNK, _CH)
    items = _mlp(x, W1, b1, W2, b2)
    return _segsum(items, idx3d)


# speedup vs baseline: 2.3813x; 1.6044x over previous
import functools

import jax
import jax.numpy as jnp
from jax import lax
from jax.experimental import pallas as pl
from jax.experimental.pallas import tpu as pltpu
from jax.experimental.pallas import tpu_sc as plsc

N = 160000
D_IN = 256
D_HID = 512
N_BINS = 10000


_ROWS_BLK = 8000


def _mlp_body(x_ref, w1_ref, b1_ref, w2_ref, b2_ref, o_ref):
    h = jnp.dot(x_ref[...], w1_ref[...], preferred_element_type=jnp.float32)
    h = jnp.maximum(h + b1_ref[...], 0.0)
    y = jnp.dot(h, w2_ref[...], preferred_element_type=jnp.float32)
    o_ref[...] = y + b2_ref[...]


def _mlp(x, W1, b1, W2, b2):
    grid = (N // _ROWS_BLK,)
    return pl.pallas_call(
        _mlp_body,
        grid=grid,
        in_specs=[
            pl.BlockSpec((_ROWS_BLK, D_IN), lambda i: (i, 0)),
            pl.BlockSpec((D_IN, D_HID), lambda i: (0, 0)),
            pl.BlockSpec((1, D_HID), lambda i: (0, 0)),
            pl.BlockSpec((D_HID, D_IN), lambda i: (0, 0)),
            pl.BlockSpec((1, D_IN), lambda i: (0, 0)),
        ],
        out_specs=pl.BlockSpec((_ROWS_BLK, D_IN), lambda i: (i, 0)),
        out_shape=jax.ShapeDtypeStruct((N, D_IN), jnp.float32),
    )(x, W1, b1.reshape(1, D_HID), W2, b2.reshape(1, D_IN))


_NC, _NS = 2, 16
_HALF = D_IN // _NC
_ROWS_PER_SUB = N // _NS
_CH = 80
_NCHUNK = _ROWS_PER_SUB // _CH
_EXP_CH = 80
_N_EXP_CHUNKS = N_BINS // _EXP_CH


def _segsum(items, idx3d):
    mesh = plsc.VectorSubcoreMesh(
        core_axis_name="c", subcore_axis_name="s",
        num_cores=_NC, num_subcores=_NS,
    )

    @functools.partial(
        pl.kernel,
        out_type=jax.ShapeDtypeStruct((N_BINS, D_IN), jnp.float32),
        mesh=mesh,
        scratch_types=[
            pltpu.VMEM((_NCHUNK, _CH), jnp.int32),
            pltpu.VMEM((_CH, _HALF), jnp.float32),
            pltpu.VMEM((_CH, _HALF), jnp.float32),
            pltpu.VMEM((_EXP_CH, _HALF), jnp.float32),
            pltpu.VMEM_SHARED((N_BINS, _HALF), jnp.float32),
            pltpu.SemaphoreType.DMA,
            pltpu.SemaphoreType.DMA,
        ],
    )
    def k(items_hbm, idx_hbm, out_hbm,
          idx_v, rows0, rows1, stage_v, acc_sh, sem0, sem1):
        c = lax.axis_index("c")
        s = lax.axis_index("s")
        col0 = c * _HALF
        row_base = s * _ROWS_PER_SUB

        pltpu.sync_copy(idx_hbm.at[s], idx_v)

        zero = jnp.zeros((16,), jnp.float32)

        def zst(i, carry):
            for j in range(_HALF // 16):
                stage_v[i, pl.ds(j * 16, 16)] = zero
            return carry

        lax.fori_loop(0, _EXP_CH, zst, 0)

        n_t = (_N_EXP_CHUNKS - s + _NS - 1) // _NS

        def zacc(t, carry):
            r0 = (s + t * _NS) * _EXP_CH
            pltpu.sync_copy(stage_v, acc_sh.at[pl.ds(r0, _EXP_CH)])
            return carry

        lax.fori_loop(0, n_t, zacc, 0)
        plsc.subcore_barrier()

        def start(chunk, buf, sem):
            row0 = row_base + chunk * _CH
            pltpu.async_copy(
                items_hbm.at[pl.ds(row0, _CH), pl.ds(col0, _HALF)], buf, sem)

        def wait(buf, sem):
            pltpu.make_async_copy(
                items_hbm.at[pl.ds(row_base, _CH), pl.ds(col0, _HALF)],
                buf, sem).wait()

        def scat(chunk, buf):
            pltpu.sync_copy(buf, acc_sh.at[idx_v.at[chunk]], add=True)

        start(0, rows0, sem0)

        def pair(i, carry):
            c0 = 2 * i
            c1 = c0 + 1
            start(c1, rows1, sem1)
            wait(rows0, sem0)
            scat(c0, rows0)

            @pl.when(c1 + 1 < _NCHUNK)
            def _():
                start(c1 + 1, rows0, sem0)

            wait(rows1, sem1)
            scat(c1, rows1)
            return carry

        lax.fori_loop(0, _NCHUNK // 2, pair, 0)
        wait(rows0, sem0)
        scat(_NCHUNK - 1, rows0)
        plsc.subcore_barrier()

        def export(t, carry):
            r0 = (s + t * _NS) * _EXP_CH
            pltpu.sync_copy(acc_sh.at[pl.ds(r0, _EXP_CH)], stage_v)
            pltpu.sync_copy(
                stage_v, out_hbm.at[pl.ds(r0, _EXP_CH), pl.ds(col0, _HALF)])
            return carry

        lax.fori_loop(0, n_t, export, 0)

    return k(items, idx3d)


def kernel(x, idxs, n_bins, W1, b1, W2, b2):
    idx32 = jnp.minimum(idxs, N_BINS - 1).astype(jnp.int32)
    idx3d = idx32.reshape(_NS, _NCHUNK, _CH)
    items = _mlp(x, W1, b1, W2, b2)
    return _segsum(items, idx3d)
